# baseline (speedup 1.0000x reference)
import functools
import math

import jax
import jax.numpy as jnp
from jax import lax
from jax.experimental import pallas as pl
from jax.experimental.pallas import tpu as pltpu
from jax.experimental.pallas import tpu_sc as plsc

S = 64.0
CC = 6400
REM = 4000


def _sc_scale_body(rpw, V, x_hbm, labels_hbm, out_hbm, bin0, bout0, sem):
    nc = 2
    wid = lax.axis_index("s") * nc + lax.axis_index("c")
    base = wid * rpw
    ngroups = rpw // 8
    nmain = V // CC

    def scale_rows(width, nvec, row0, off, bin_r, bout_r):
        pltpu.sync_copy(x_hbm.at[pl.ds(row0, 8), pl.ds(off, width)], bin_r)

        def vec_fn(j, _):
            o = pl.multiple_of(j * 16, 16)
            for r in range(8):
                bout_r[r, pl.ds(o, 16)] = bin_r[r, pl.ds(o, 16)] * S
            return 0

        lax.fori_loop(0, nvec, vec_fn, 0)
        pltpu.sync_copy(bout_r, out_hbm.at[pl.ds(row0, 8), pl.ds(off, width)])

    def chunk_fn(i, carry):
        rg = i // nmain
        ch = i % nmain
        row0 = pl.multiple_of(base + rg * 8, 8)
        off = pl.multiple_of(ch * CC, 128)
        scale_rows(CC, CC // 16, row0, off, bin0, bout0)
        return carry

    lax.fori_loop(0, ngroups * nmain, chunk_fn, 0)


def kernel(logits, labels):
    B, V = logits.shape
    mesh = plsc.VectorSubcoreMesh(core_axis_name="c", subcore_axis_name="s")
    nw = mesh.num_cores * mesh.num_subcores
    rpw = B // nw
    return pl.kernel(
        functools.partial(_sc_scale_body, rpw, V),
        out_type=jax.ShapeDtypeStruct((B, V), jnp.float32),
        mesh=mesh,
        compiler_params=pltpu.CompilerParams(needs_layout_passes=False),
        scratch_types=[
            pltpu.VMEM((8, CC), jnp.float32),
            pltpu.VMEM((8, CC), jnp.float32),
            pltpu.SemaphoreType.DMA,
        ],
    )(logits, labels)


# speedup vs baseline: 1.3422x; 1.3422x over previous
import functools
import math

import jax
import jax.numpy as jnp
from jax import lax
from jax.experimental import pallas as pl
from jax.experimental.pallas import tpu as pltpu
from jax.experimental.pallas import tpu_sc as plsc

S = 64.0
MARGIN = 0.5
COS_M = math.cos(MARGIN)
SIN_M = math.sin(MARGIN)

ROW_BLOCK = 16


def _sc_gather_body(rpw, V, logits_hbm, labels_hbm, t_hbm, lab_v, win_v, t_v, sem):
    nc = 2
    wid = lax.axis_index("s") * nc + lax.axis_index("c")
    base = wid * rpw
    pltpu.sync_copy(labels_hbm.at[pl.ds(base, rpw)], lab_v)
    lab_regs = [lab_v[pl.ds(k * 16, 16)] for k in range(rpw // 16)]
    copies = []
    for j in range(rpw):
        lab_j = jnp.maximum(lab_regs[j // 16][j % 16], 0)
        start_j = jnp.minimum((lab_j // 16) * 16, V - 16)
        c = pltpu.make_async_copy(
            logits_hbm.at[base + j, pl.ds(start_j, 16)],
            win_v.at[pl.ds(j * 16, 16)],
            sem,
        )
        c.start()
        copies.append(c)
    for c in copies:
        c.wait()
    iota = lax.iota(jnp.int32, 16)
    for k in range(rpw // 16):
        lab_vec = jnp.maximum(lab_regs[k], 0)
        start_vec = jnp.minimum((lab_vec // 16) * 16, V - 16)
        off = (k * 16 + iota) * 16 + (lab_vec - start_vec)
        t_v[pl.ds(k * 16, 16)] = plsc.load_gather(win_v, [off])
    pltpu.sync_copy(t_v, t_hbm.at[pl.ds(base, rpw)])


def _sc_gather(logits, labels):
    B, V = logits.shape
    mesh = plsc.VectorSubcoreMesh(core_axis_name="c", subcore_axis_name="s")
    nw = mesh.num_cores * mesh.num_subcores
    rpw = B // nw
    return pl.kernel(
        functools.partial(_sc_gather_body, rpw, V),
        out_type=jax.ShapeDtypeStruct((B,), jnp.float32),
        mesh=mesh,
        compiler_params=pltpu.CompilerParams(needs_layout_passes=False),
        scratch_types=[
            pltpu.VMEM((rpw,), jnp.int32),
            pltpu.VMEM((rpw * 16,), jnp.float32),
            pltpu.VMEM((rpw,), jnp.float32),
            pltpu.SemaphoreType.DMA,
        ],
    )(logits, labels)


def _arcface_block(labels_smem, t_smem, x_ref, out_ref):
    i = pl.program_id(0)
    out_ref[...] = x_ref[...] * S
    for r in range(ROW_BLOCK):
        row = i * ROW_BLOCK + r
        lab = labels_smem[row]
        t = jnp.full((1, 1), t_smem[row], jnp.float32)
        spec = (t * COS_M - jnp.sqrt(jnp.maximum(1.0 - t * t, 0.0)) * SIN_M) * S

        @pl.when(lab >= 0)
        def _():
            col0 = pl.multiple_of((lab // 128) * 128, 128)
            chunk = out_ref[pl.ds(r, 1), pl.ds(col0, 128)]
            m = jax.lax.broadcasted_iota(jnp.int32, (1, 128), 1) == lab - col0
            out_ref[pl.ds(r, 1), pl.ds(col0, 128)] = jnp.where(m, spec, chunk)


def kernel(logits, labels):
    B, V = logits.shape
    t = _sc_gather(logits, labels)
    grid = (B // ROW_BLOCK,)
    return pl.pallas_call(
        _arcface_block,
        grid=grid,
        in_specs=[
            pl.BlockSpec(memory_space=pltpu.SMEM),
            pl.BlockSpec(memory_space=pltpu.SMEM),
            pl.BlockSpec((ROW_BLOCK, V), lambda i: (i, 0)),
        ],
        out_specs=pl.BlockSpec((ROW_BLOCK, V), lambda i: (i, 0)),
        out_shape=jax.ShapeDtypeStruct((B, V), jnp.float32),
        compiler_params=pltpu.CompilerParams(
            dimension_semantics=("arbitrary",),
        ),
    )(labels, t, logits)


# speedup vs baseline: 1.3447x; 1.0019x over previous
import functools
import math

import jax
import jax.numpy as jnp
from jax import lax
from jax.experimental import pallas as pl
from jax.experimental.pallas import tpu as pltpu
from jax.experimental.pallas import tpu_sc as plsc

S = 64.0
MARGIN = 0.5
COS_M = math.cos(MARGIN)
SIN_M = math.sin(MARGIN)

ROW_BLOCK = 32


def _sc_gather_body(rpw, V, logits_hbm, labels_hbm, t_hbm, lab_v, win_v, t_v, sem):
    nc = 2
    wid = lax.axis_index("s") * nc + lax.axis_index("c")
    base = wid * rpw
    pltpu.sync_copy(labels_hbm.at[pl.ds(base, rpw)], lab_v)
    lab_regs = [lab_v[pl.ds(k * 16, 16)] for k in range(rpw // 16)]
    copies = []
    for j in range(rpw):
        lab_j = jnp.maximum(lab_regs[j // 16][j % 16], 0)
        start_j = jnp.minimum((lab_j // 16) * 16, V - 16)
        c = pltpu.make_async_copy(
            logits_hbm.at[base + j, pl.ds(start_j, 16)],
            win_v.at[pl.ds(j * 16, 16)],
            sem,
        )
        c.start()
        copies.append(c)
    for c in copies:
        c.wait()
    iota = lax.iota(jnp.int32, 16)
    for k in range(rpw // 16):
        lab_vec = jnp.maximum(lab_regs[k], 0)
        start_vec = jnp.minimum((lab_vec // 16) * 16, V - 16)
        off = (k * 16 + iota) * 16 + (lab_vec - start_vec)
        t_v[pl.ds(k * 16, 16)] = plsc.load_gather(win_v, [off])
    pltpu.sync_copy(t_v, t_hbm.at[pl.ds(base, rpw)])


def _sc_gather(logits, labels):
    B, V = logits.shape
    mesh = plsc.VectorSubcoreMesh(core_axis_name="c", subcore_axis_name="s")
    nw = mesh.num_cores * mesh.num_subcores
    rpw = B // nw
    return pl.kernel(
        functools.partial(_sc_gather_body, rpw, V),
        out_type=jax.ShapeDtypeStruct((B,), jnp.float32),
        mesh=mesh,
        compiler_params=pltpu.CompilerParams(needs_layout_passes=False),
        scratch_types=[
            pltpu.VMEM((rpw,), jnp.int32),
            pltpu.VMEM((rpw * 16,), jnp.float32),
            pltpu.VMEM((rpw,), jnp.float32),
            pltpu.SemaphoreType.DMA,
        ],
    )(logits, labels)


def _arcface_block(labels_smem, t_smem, x_ref, out_ref):
    i = pl.program_id(0)
    out_ref[...] = x_ref[...] * S
    for r in range(ROW_BLOCK):
        row = i * ROW_BLOCK + r
        lab = labels_smem[row]
        t = jnp.full((1, 1), t_smem[row], jnp.float32)
        spec = (t * COS_M - jnp.sqrt(jnp.maximum(1.0 - t * t, 0.0)) * SIN_M) * S

        @pl.when(lab >= 0)
        def _():
            col0 = pl.multiple_of((lab // 128) * 128, 128)
            chunk = out_ref[pl.ds(r, 1), pl.ds(col0, 128)]
            m = jax.lax.broadcasted_iota(jnp.int32, (1, 128), 1) == lab - col0
            out_ref[pl.ds(r, 1), pl.ds(col0, 128)] = jnp.where(m, spec, chunk)


def kernel(logits, labels):
    B, V = logits.shape
    t = _sc_gather(logits, labels)
    grid = (B // ROW_BLOCK,)
    return pl.pallas_call(
        _arcface_block,
        grid=grid,
        in_specs=[
            pl.BlockSpec(memory_space=pltpu.SMEM),
            pl.BlockSpec(memory_space=pltpu.SMEM),
            pl.BlockSpec((ROW_BLOCK, V), lambda i: (i, 0)),
        ],
        out_specs=pl.BlockSpec((ROW_BLOCK, V), lambda i: (i, 0)),
        out_shape=jax.ShapeDtypeStruct((B, V), jnp.float32),
        compiler_params=pltpu.CompilerParams(
            dimension_semantics=("arbitrary",),
        ),
    )(labels, t, logits)


# speedup vs baseline: 1.3573x; 1.0094x over previous
import math

import jax
import jax.numpy as jnp
from jax.experimental import pallas as pl
from jax.experimental.pallas import tpu as pltpu

S = 64.0
MARGIN = 0.5
COS_M = math.cos(MARGIN)
SIN_M = math.sin(MARGIN)

ROW_BLOCK = 32


def _arcface_block(labels_smem, x_ref, out_ref):
    i = pl.program_id(0)
    out_ref[...] = x_ref[...] * S
    for r in range(ROW_BLOCK):
        row = i * ROW_BLOCK + r
        lab = labels_smem[row]

        @pl.when(lab >= 0)
        def _():
            col0 = pl.multiple_of((lab // 128) * 128, 128)
            chunk = x_ref[pl.ds(r, 1), pl.ds(col0, 128)]
            m = jax.lax.broadcasted_iota(jnp.int32, (1, 128), 1) == lab - col0
            t = jnp.sum(jnp.where(m, chunk, 0.0))
            spec = (t * COS_M - jnp.sqrt(jnp.maximum(1.0 - t * t, 0.0)) * SIN_M) * S
            out_ref[pl.ds(r, 1), pl.ds(col0, 128)] = jnp.where(m, spec, chunk * S)


def kernel(logits, labels):
    B, V = logits.shape
    grid = (B // ROW_BLOCK,)
    return pl.pallas_call(
        _arcface_block,
        grid=grid,
        in_specs=[
            pl.BlockSpec(memory_space=pltpu.SMEM),
            pl.BlockSpec((ROW_BLOCK, V), lambda i: (i, 0)),
        ],
        out_specs=pl.BlockSpec((ROW_BLOCK, V), lambda i: (i, 0)),
        out_shape=jax.ShapeDtypeStruct((B, V), jnp.float32),
        compiler_params=pltpu.CompilerParams(
            dimension_semantics=("arbitrary",),
        ),
    )(labels, logits)
